# baseline (speedup 1.0000x reference)
import functools

import jax
import jax.numpy as jnp
from jax import lax
from jax.experimental import pallas as pl
from jax.experimental.pallas import tpu as pltpu
from jax.experimental.pallas import tpu_sc as plsc

IN_F = 1024
OUT_F = 2048
E = 16
BM = 128

_SC_INFO = plsc.get_sparse_core_info()
_NC = _SC_INFO.num_cores
_NS = _SC_INFO.num_subcores
_NW = _NC * _NS


_NBUF = 4
_LOOK = 2


@functools.partial(jax.jit, static_argnames=("chunk",))
def _sc_row_gather(table, idx, chunk):
    B = idx.shape[0]
    D = table.shape[1]
    dtype = table.dtype
    b_per_w = B // _NW
    nchunks = b_per_w // chunk
    assert b_per_w % chunk == 0 and nchunks >= _NBUF
    mesh = plsc.VectorSubcoreMesh(core_axis_name="c", subcore_axis_name="s")

    @functools.partial(
        pl.kernel, mesh=mesh,
        out_type=jax.ShapeDtypeStruct((B, D), dtype),
        scratch_types=[pltpu.VMEM((b_per_w,), jnp.int32)]
        + [pltpu.VMEM((chunk, D), dtype) for _ in range(_NBUF)]
        + [pltpu.SemaphoreType.DMA for _ in range(2 * _NBUF)],
    )
    def k(table_hbm, idx_hbm, out_hbm, idx_v, *bufs_sems):
        rows = bufs_sems[:_NBUF]
        gs = bufs_sems[_NBUF:2 * _NBUF]
        ws = bufs_sems[2 * _NBUF:]
        wid = lax.axis_index("s") * _NC + lax.axis_index("c")
        base = wid * b_per_w
        pltpu.sync_copy(idx_hbm.at[pl.ds(base, b_per_w)], idx_v)

        def gather(c, b):
            sl = idx_v.at[pl.ds(pl.multiple_of(c * chunk, chunk), chunk)]
            return pltpu.make_async_copy(table_hbm.at[sl], rows[b], gs[b])

        def wback(c, b):
            return pltpu.make_async_copy(
                rows[b], out_hbm.at[pl.ds(base + c * chunk, chunk)], ws[b])

        for b in range(_LOOK):
            gather(b, b).start()

        def step(c, b_g, b_w):
            @pl.when(c + _LOOK < nchunks)
            def _():
                @pl.when(c + _LOOK >= _NBUF)
                def _():
                    wback(c + _LOOK - _NBUF, b_g).wait()
                gather(c + _LOOK, b_g).start()
            gather(c, b_w).wait()
            wback(c, b_w).start()

        def outer(i, _):
            c0 = i * _NBUF
            for b in range(_NBUF):
                step(c0 + b, (b + _LOOK) % _NBUF, b)
            return ()

        lax.fori_loop(0, nchunks // _NBUF, outer, (), unroll=False)
        for c in range(nchunks - _NBUF, nchunks):
            wback(c, c % _NBUF).wait()

    return k(table, idx)


@functools.partial(jax.jit, static_argnames=("P", "chunk"))
def _sc_row_scatter(table, idx3, P, chunk):
    B, D = table.shape
    dtype = table.dtype
    b_per_w = B // _NW
    nchunks = b_per_w // chunk
    assert b_per_w % chunk == 0 and nchunks >= _NBUF
    mesh = plsc.VectorSubcoreMesh(core_axis_name="c", subcore_axis_name="s")

    @functools.partial(
        pl.kernel, mesh=mesh,
        out_type=jax.ShapeDtypeStruct((P, D), dtype),
        scratch_types=[pltpu.VMEM((nchunks, chunk), jnp.int32)]
        + [pltpu.VMEM((chunk, D), dtype) for _ in range(_NBUF)]
        + [pltpu.SemaphoreType.DMA for _ in range(2 * _NBUF)],
    )
    def k(table_hbm, idx_hbm, out_hbm, idx_v, *bufs_sems):
        rows = bufs_sems[:_NBUF]
        gs = bufs_sems[_NBUF:2 * _NBUF]
        ws = bufs_sems[2 * _NBUF:]
        wid = lax.axis_index("s") * _NC + lax.axis_index("c")
        base = wid * b_per_w
        pltpu.sync_copy(idx_hbm.at[wid], idx_v)

        def load(c, b):
            return pltpu.make_async_copy(
                table_hbm.at[pl.ds(base + c * chunk, chunk)], rows[b], gs[b])

        def scat(c, b):
            return pltpu.make_async_copy(
                rows[b], out_hbm.at[idx_v.at[c]], ws[b])

        for b in range(_LOOK):
            load(b, b).start()

        def step(c, b_g, b_w):
            @pl.when(c + _LOOK < nchunks)
            def _():
                @pl.when(c + _LOOK >= _NBUF)
                def _():
                    scat(c + _LOOK - _NBUF, b_g).wait()
                load(c + _LOOK, b_g).start()
            load(c, b_w).wait()
            scat(c, b_w).start()

        def outer(i, _):
            c0 = i * _NBUF
            for b in range(_NBUF):
                step(c0 + b, (b + _LOOK) % _NBUF, b)
            return ()

        lax.fori_loop(0, nchunks // _NBUF, outer, (), unroll=False)
        for c in range(nchunks - _NBUF, nchunks):
            scat(c, c % _NBUF).wait()

    return k(table, idx3)


def _pack_block(x_ref, o_ref):
    res = x_ref[...].astype(jnp.bfloat16)
    lo = lax.bitcast_convert_type(res[:, :IN_F // 2], jnp.uint16)
    hi = lax.bitcast_convert_type(res[:, IN_F // 2:], jnp.uint16)
    packed = lo.astype(jnp.uint32) | (hi.astype(jnp.uint32) << 16)
    o_ref[...] = lax.bitcast_convert_type(packed, jnp.int32)


@jax.jit
def _pack_x(xf):
    T = xf.shape[0]
    blk = 512
    return pl.pallas_call(
        _pack_block,
        grid=(T // blk,),
        in_specs=[pl.BlockSpec((blk, IN_F), lambda i: (i, 0))],
        out_specs=pl.BlockSpec((blk, IN_F // 2), lambda i: (i, 0)),
        out_shape=jax.ShapeDtypeStruct((T, IN_F // 2), jnp.int32),
    )(xf)


def _unpack_bf16(p):
    pu = lax.bitcast_convert_type(p, jnp.uint32)
    lo = lax.bitcast_convert_type((pu & 0xFFFF).astype(jnp.uint16),
                                  jnp.bfloat16)
    hi = lax.bitcast_convert_type((pu >> 16).astype(jnp.uint16), jnp.bfloat16)
    return lo, hi


def _gemm_block(be_ref, x_ref, w_ref, b_ref, o_ref):
    xlo, xhi = _unpack_bf16(x_ref[...])
    xb = jnp.concatenate([xlo, xhi], axis=1)
    acc = jax.lax.dot_general(
        xb, w_ref[0],
        dimension_numbers=(((1,), (1,)), ((), ())),
        preferred_element_type=jnp.float32,
    )
    res = acc.astype(jnp.bfloat16) + b_ref[0]
    lo = lax.bitcast_convert_type(res[:, :OUT_F // 2], jnp.uint16)
    hi = lax.bitcast_convert_type(res[:, OUT_F // 2:], jnp.uint16)
    packed = lo.astype(jnp.uint32) | (hi.astype(jnp.uint32) << 16)
    o_ref[...] = lax.bitcast_convert_type(packed, jnp.int32)


@functools.partial(jax.jit, static_argnames=("nblocks",))
def _grouped_gemm(x_s, w_b, bias_b, block_expert, nblocks):
    grid_spec = pltpu.PrefetchScalarGridSpec(
        num_scalar_prefetch=1,
        grid=(nblocks,),
        in_specs=[
            pl.BlockSpec((BM, IN_F // 2), lambda i, be: (i, 0)),
            pl.BlockSpec((1, OUT_F, IN_F), lambda i, be: (be[i], 0, 0)),
            pl.BlockSpec((1, 1, OUT_F), lambda i, be: (be[i], 0, 0)),
        ],
        out_specs=pl.BlockSpec((BM, OUT_F // 2), lambda i, be: (i, 0)),
    )
    return pl.pallas_call(
        _gemm_block,
        grid_spec=grid_spec,
        out_shape=jax.ShapeDtypeStruct((nblocks * BM, OUT_F // 2), jnp.int32),
    )(block_expert, x_s, w_b, bias_b)


def kernel(x, weight, bias, ids):
    out_shape = x.shape[:-1] + (OUT_F,)
    T = x.shape[0] * x.shape[1]
    P = T + E * BM
    NB = P // BM

    xf = x.reshape(T, IN_F)
    idf = ids.reshape(T)

    oh = (idf[None, :] == jnp.arange(E, dtype=jnp.int32)[:, None]
          ).astype(jnp.int32)
    occ = jnp.cumsum(oh, axis=1)
    counts = occ[:, -1]
    rank = jnp.sum(oh * occ, axis=0) - 1
    padded = ((counts + BM - 1) // BM) * BM
    cum_padded = jnp.cumsum(padded)
    p_off = cum_padded - padded
    pos = p_off[idf] + rank
    blk_start = jnp.arange(NB, dtype=jnp.int32) * BM
    block_expert = jnp.minimum(
        jnp.sum(blk_start[:, None] >= cum_padded[None, :], axis=1,
                dtype=jnp.int32), E - 1)

    x_packed = _pack_x(xf)
    pos3 = pos.reshape(_NW, (T // _NW) // 32, 32)
    x_s = _sc_row_scatter(x_packed, pos3, P, chunk=32)

    w_b = weight.astype(jnp.bfloat16)
    bias_b = bias.astype(jnp.bfloat16).reshape(E, 1, OUT_F)

    out_s = _grouped_gemm(x_s, w_b, bias_b, block_expert, NB)

    out_i = _sc_row_gather(out_s, pos, chunk=16)

    lo, hi = _unpack_bf16(out_i)
    out = jnp.concatenate([lo, hi], axis=-1)
    return out.reshape(out_shape)


# speedup vs baseline: 1.1832x; 1.1832x over previous
import functools

import jax
import jax.numpy as jnp
from jax import lax
from jax.experimental import pallas as pl
from jax.experimental.pallas import tpu as pltpu
from jax.experimental.pallas import tpu_sc as plsc

IN_F = 1024
OUT_F = 2048
E = 16
BM = 256

_SC_INFO = plsc.get_sparse_core_info()
_NC = _SC_INFO.num_cores
_NS = _SC_INFO.num_subcores
_NW = _NC * _NS


_NBUF = 4
_LOOK = 2


@functools.partial(jax.jit, static_argnames=("chunk",))
def _sc_row_gather(table, idx, chunk):
    B = idx.shape[0]
    D = table.shape[1]
    dtype = table.dtype
    b_per_w = B // _NW
    nchunks = b_per_w // chunk
    assert b_per_w % chunk == 0 and nchunks >= _NBUF
    mesh = plsc.VectorSubcoreMesh(core_axis_name="c", subcore_axis_name="s")

    @functools.partial(
        pl.kernel, mesh=mesh,
        out_type=jax.ShapeDtypeStruct((B, D), dtype),
        scratch_types=[pltpu.VMEM((b_per_w,), jnp.int32)]
        + [pltpu.VMEM((chunk, D), dtype) for _ in range(_NBUF)]
        + [pltpu.SemaphoreType.DMA for _ in range(2 * _NBUF)],
    )
    def k(table_hbm, idx_hbm, out_hbm, idx_v, *bufs_sems):
        rows = bufs_sems[:_NBUF]
        gs = bufs_sems[_NBUF:2 * _NBUF]
        ws = bufs_sems[2 * _NBUF:]
        wid = lax.axis_index("s") * _NC + lax.axis_index("c")
        base = wid * b_per_w
        pltpu.sync_copy(idx_hbm.at[pl.ds(base, b_per_w)], idx_v)

        def gather(c, b):
            sl = idx_v.at[pl.ds(pl.multiple_of(c * chunk, chunk), chunk)]
            return pltpu.make_async_copy(table_hbm.at[sl], rows[b], gs[b])

        def wback(c, b):
            return pltpu.make_async_copy(
                rows[b], out_hbm.at[pl.ds(base + c * chunk, chunk)], ws[b])

        for b in range(_LOOK):
            gather(b, b).start()

        def step(c, b_g, b_w):
            @pl.when(c + _LOOK < nchunks)
            def _():
                @pl.when(c + _LOOK >= _NBUF)
                def _():
                    wback(c + _LOOK - _NBUF, b_g).wait()
                gather(c + _LOOK, b_g).start()
            gather(c, b_w).wait()
            wback(c, b_w).start()

        def outer(i, _):
            c0 = i * _NBUF
            for b in range(_NBUF):
                step(c0 + b, (b + _LOOK) % _NBUF, b)
            return ()

        lax.fori_loop(0, nchunks // _NBUF, outer, (), unroll=False)
        for c in range(nchunks - _NBUF, nchunks):
            wback(c, c % _NBUF).wait()

    return k(table, idx)


@functools.partial(jax.jit, static_argnames=("P", "chunk"))
def _sc_row_scatter(table, idx3, P, chunk):
    B, D = table.shape
    dtype = table.dtype
    b_per_w = B // _NW
    nchunks = b_per_w // chunk
    assert b_per_w % chunk == 0 and nchunks >= _NBUF
    mesh = plsc.VectorSubcoreMesh(core_axis_name="c", subcore_axis_name="s")

    @functools.partial(
        pl.kernel, mesh=mesh,
        out_type=jax.ShapeDtypeStruct((P, D), dtype),
        scratch_types=[pltpu.VMEM((nchunks, chunk), jnp.int32)]
        + [pltpu.VMEM((chunk, D), dtype) for _ in range(_NBUF)]
        + [pltpu.SemaphoreType.DMA for _ in range(2 * _NBUF)],
    )
    def k(table_hbm, idx_hbm, out_hbm, idx_v, *bufs_sems):
        rows = bufs_sems[:_NBUF]
        gs = bufs_sems[_NBUF:2 * _NBUF]
        ws = bufs_sems[2 * _NBUF:]
        wid = lax.axis_index("s") * _NC + lax.axis_index("c")
        base = wid * b_per_w
        pltpu.sync_copy(idx_hbm.at[wid], idx_v)

        def load(c, b):
            return pltpu.make_async_copy(
                table_hbm.at[pl.ds(base + c * chunk, chunk)], rows[b], gs[b])

        def scat(c, b):
            return pltpu.make_async_copy(
                rows[b], out_hbm.at[idx_v.at[c]], ws[b])

        for b in range(_LOOK):
            load(b, b).start()

        def step(c, b_g, b_w):
            @pl.when(c + _LOOK < nchunks)
            def _():
                @pl.when(c + _LOOK >= _NBUF)
                def _():
                    scat(c + _LOOK - _NBUF, b_g).wait()
                load(c + _LOOK, b_g).start()
            load(c, b_w).wait()
            scat(c, b_w).start()

        def outer(i, _):
            c0 = i * _NBUF
            for b in range(_NBUF):
                step(c0 + b, (b + _LOOK) % _NBUF, b)
            return ()

        lax.fori_loop(0, nchunks // _NBUF, outer, (), unroll=False)
        for c in range(nchunks - _NBUF, nchunks):
            scat(c, c % _NBUF).wait()

    return k(table, idx3)


def _pack_block(x_ref, o_ref):
    res = x_ref[...].astype(jnp.bfloat16)
    lo = lax.bitcast_convert_type(res[:, :IN_F // 2], jnp.uint16)
    hi = lax.bitcast_convert_type(res[:, IN_F // 2:], jnp.uint16)
    packed = lo.astype(jnp.uint32) | (hi.astype(jnp.uint32) << 16)
    o_ref[...] = lax.bitcast_convert_type(packed, jnp.int32)


@jax.jit
def _pack_x(xf):
    T = xf.shape[0]
    blk = 512
    return pl.pallas_call(
        _pack_block,
        grid=(T // blk,),
        in_specs=[pl.BlockSpec((blk, IN_F), lambda i: (i, 0))],
        out_specs=pl.BlockSpec((blk, IN_F // 2), lambda i: (i, 0)),
        out_shape=jax.ShapeDtypeStruct((T, IN_F // 2), jnp.int32),
    )(xf)


def _unpack_bf16(p):
    pu = lax.bitcast_convert_type(p, jnp.uint32)
    lo = lax.bitcast_convert_type((pu & 0xFFFF).astype(jnp.uint16),
                                  jnp.bfloat16)
    hi = lax.bitcast_convert_type((pu >> 16).astype(jnp.uint16), jnp.bfloat16)
    return lo, hi


def _gemm_block(be_ref, x_ref, w_ref, b_ref, o_ref):
    xlo, xhi = _unpack_bf16(x_ref[...])
    dn = (((1,), (1,)), ((), ()))
    acc = jax.lax.dot_general(
        xlo, w_ref[0, :, :IN_F // 2], dimension_numbers=dn,
        preferred_element_type=jnp.float32,
    ) + jax.lax.dot_general(
        xhi, w_ref[0, :, IN_F // 2:], dimension_numbers=dn,
        preferred_element_type=jnp.float32,
    )
    res = acc.astype(jnp.bfloat16) + b_ref[0]
    lo = lax.bitcast_convert_type(res[:, :OUT_F // 2], jnp.uint16)
    hi = lax.bitcast_convert_type(res[:, OUT_F // 2:], jnp.uint16)
    packed = lo.astype(jnp.uint32) | (hi.astype(jnp.uint32) << 16)
    o_ref[...] = lax.bitcast_convert_type(packed, jnp.int32)


@functools.partial(jax.jit, static_argnames=("nblocks",))
def _grouped_gemm(x_s, w_b, bias_b, block_expert, nblocks):
    grid_spec = pltpu.PrefetchScalarGridSpec(
        num_scalar_prefetch=1,
        grid=(nblocks,),
        in_specs=[
            pl.BlockSpec((BM, IN_F // 2), lambda i, be: (i, 0)),
            pl.BlockSpec((1, OUT_F, IN_F), lambda i, be: (be[i], 0, 0)),
            pl.BlockSpec((1, 1, OUT_F), lambda i, be: (be[i], 0, 0)),
        ],
        out_specs=pl.BlockSpec((BM, OUT_F // 2), lambda i, be: (i, 0)),
    )
    return pl.pallas_call(
        _gemm_block,
        grid_spec=grid_spec,
        out_shape=jax.ShapeDtypeStruct((nblocks * BM, OUT_F // 2), jnp.int32),
    )(block_expert, x_s, w_b, bias_b)


def kernel(x, weight, bias, ids):
    out_shape = x.shape[:-1] + (OUT_F,)
    T = x.shape[0] * x.shape[1]
    P = T + E * BM
    NB = P // BM

    xf = x.reshape(T, IN_F)
    idf = ids.reshape(T)

    oh = (idf[None, :] == jnp.arange(E, dtype=jnp.int32)[:, None]
          ).astype(jnp.int32)
    occ = jnp.cumsum(oh, axis=1)
    counts = occ[:, -1]
    rank = jnp.sum(oh * occ, axis=0) - 1
    padded = ((counts + BM - 1) // BM) * BM
    cum_padded = jnp.cumsum(padded)
    p_off = cum_padded - padded
    pos = p_off[idf] + rank
    blk_start = jnp.arange(NB, dtype=jnp.int32) * BM
    block_expert = jnp.minimum(
        jnp.sum(blk_start[:, None] >= cum_padded[None, :], axis=1,
                dtype=jnp.int32), E - 1)

    x_packed = _pack_x(xf)
    pos3 = pos.reshape(_NW, (T // _NW) // 32, 32)
    x_s = _sc_row_scatter(x_packed, pos3, P, chunk=32)

    w_b = weight.astype(jnp.bfloat16)
    bias_b = bias.astype(jnp.bfloat16).reshape(E, 1, OUT_F)

    out_s = _grouped_gemm(x_s, w_b, bias_b, block_expert, NB)

    out_i = _sc_row_gather(out_s, pos, chunk=16)

    lo, hi = _unpack_bf16(out_i)
    out = jnp.concatenate([lo, hi], axis=-1)
    return out.reshape(out_shape)
